# Initial kernel scaffold; baseline (speedup 1.0000x reference)
import jax
import jax.numpy as jnp
from jax.experimental import pallas as pl


def kernel(x, edge_index, W1, b1, W_mu, b_mu, W_ls, b_ls):
    raise NotImplementedError("write your pallas kernel here")


# speedup vs baseline: 14.4305x; 14.4305x over previous
import functools

import jax
import jax.numpy as jnp
from jax import lax
from jax.experimental import pallas as pl
from jax.experimental.pallas import tpu as pltpu
from jax.experimental.pallas import tpu_sc as plsc

NC = 2
NS = 16
NW = NC * NS


def _chunk_size(epw):
    for k in (128, 80, 64, 40, 16, 8):
        if epw % k == 0:
            return k
    raise ValueError(f"edges-per-worker {epw} not divisible by a valid chunk")


@functools.lru_cache(maxsize=None)
def _deg_pass(n_pad, e, d):
    epw = e // NW
    k = _chunk_size(epw)
    nchunk = epw // k
    rpt = n_pad // NS

    mesh = plsc.VectorSubcoreMesh(core_axis_name="c", subcore_axis_name="s")

    @functools.partial(
        pl.kernel,
        out_type=jax.ShapeDtypeStruct((NC, n_pad, d), jnp.float32),
        mesh=mesh,
        scratch_types=[
            pltpu.VMEM((k,), jnp.int32),
            pltpu.VMEM((k, d), jnp.float32),
            pltpu.VMEM_SHARED((n_pad, d), jnp.float32),
        ],
    )
    def deg_kernel(dst_hbm, ones_hbm, zeros_hbm, out_hbm, dst_v, ones_v, acc_sh):
        cid = lax.axis_index("c")
        sid = lax.axis_index("s")
        wid = cid * NS + sid

        pltpu.sync_copy(ones_hbm, ones_v)

        r0 = sid * rpt
        pltpu.sync_copy(zeros_hbm.at[pl.ds(r0, rpt)], acc_sh.at[pl.ds(r0, rpt)])
        plsc.subcore_barrier()

        @pl.loop(0, nchunk)
        def _body(ci):
            base = pl.multiple_of(wid * epw + ci * k, 8)
            pltpu.sync_copy(dst_hbm.at[pl.ds(base, k)], dst_v)
            pltpu.sync_copy(ones_v, acc_sh.at[dst_v], add=True)

        plsc.subcore_barrier()
        pltpu.sync_copy(
            acc_sh.at[pl.ds(r0, rpt)], out_hbm.at[cid, pl.ds(r0, rpt)]
        )

    return deg_kernel


@functools.lru_cache(maxsize=None)
def _edge_pass(n_pad, e, d):
    epw = e // NW
    k = _chunk_size(epw)
    nchunk = epw // k
    rpt = n_pad // NS

    mesh = plsc.VectorSubcoreMesh(core_axis_name="c", subcore_axis_name="s")

    @functools.partial(
        pl.kernel,
        out_type=jax.ShapeDtypeStruct((NC, n_pad, d), jnp.float32),
        mesh=mesh,
        scratch_types=[
            pltpu.VMEM((k,), jnp.int32),
            pltpu.VMEM((k,), jnp.int32),
            pltpu.VMEM((k, d), jnp.float32),
            pltpu.VMEM_SHARED((n_pad, d), jnp.float32),
            pltpu.SemaphoreType.DMA,
        ],
    )
    def pass_kernel(
        y_hbm, src_hbm, dst_hbm, zeros_hbm, out_hbm,
        src_v, dst_v, rows_v, acc_sh, sem,
    ):
        cid = lax.axis_index("c")
        sid = lax.axis_index("s")
        wid = cid * NS + sid

        r0 = sid * rpt
        pltpu.sync_copy(zeros_hbm.at[pl.ds(r0, rpt)], acc_sh.at[pl.ds(r0, rpt)])
        plsc.subcore_barrier()

        @pl.loop(0, nchunk)
        def _body(ci):
            base = pl.multiple_of(wid * epw + ci * k, 8)
            pltpu.sync_copy(src_hbm.at[pl.ds(base, k)], src_v)
            pltpu.sync_copy(dst_hbm.at[pl.ds(base, k)], dst_v)
            pltpu.async_copy(y_hbm.at[src_v], rows_v, sem).wait()
            pltpu.sync_copy(rows_v, acc_sh.at[dst_v], add=True)

        plsc.subcore_barrier()
        pltpu.sync_copy(
            acc_sh.at[pl.ds(r0, rpt)], out_hbm.at[cid, pl.ds(r0, rpt)]
        )

    return pass_kernel


def _dinv_block(deg_a, deg_b):
    return lax.rsqrt(deg_a[0, :, 0:1] + deg_b[0, :, 0:1] + 1.0)


def _scale_body(deg_a, deg_b, x_ref, o_ref):
    o_ref[...] = _dinv_block(deg_a[...], deg_b[...]) * x_ref[...]


def _fused1_body(deg_a, deg_b, s_a, s_b, y_ref, w_ref, b_ref, o_ref):
    dinv = _dinv_block(deg_a[...], deg_b[...])
    agg = dinv * (s_a[0] + s_b[0] + y_ref[...])
    h = (
        jnp.dot(
            agg,
            w_ref[...],
            precision=lax.Precision.HIGHEST,
            preferred_element_type=jnp.float32,
        )
        + b_ref[...]
    )
    o_ref[...] = dinv * jnp.maximum(h, 0.0)


def _fused2_body(deg_a, deg_b, s_a, s_b, y_ref, w_ref, b_ref, o_ref):
    dinv = _dinv_block(deg_a[...], deg_b[...])
    agg = dinv * (s_a[0] + s_b[0] + y_ref[...])
    o_ref[...] = (
        jnp.dot(
            agg,
            w_ref[...],
            precision=lax.Precision.HIGHEST,
            preferred_element_type=jnp.float32,
        )
        + b_ref[...]
    )


def _core_blocks(r, width):
    top = pl.BlockSpec((1, r, width), lambda i: (0, i, 0))
    bot = pl.BlockSpec((1, r, width), lambda i: (1, i, 0))
    return top, bot


def _scale_call(n, din, r, deg2, x):
    dtop, dbot = _core_blocks(r, din)
    return pl.pallas_call(
        _scale_body,
        out_shape=jax.ShapeDtypeStruct((n, din), jnp.float32),
        grid=(n // r,),
        in_specs=[
            dtop,
            dbot,
            pl.BlockSpec((r, din), lambda i: (i, 0)),
        ],
        out_specs=pl.BlockSpec((r, din), lambda i: (i, 0)),
    )(deg2, deg2, x)


def _fused_call(body, n, din, dout, r, deg2, s2, y, w, b):
    dtop, dbot = _core_blocks(r, din)
    stop, sbot = _core_blocks(r, din)
    return pl.pallas_call(
        body,
        out_shape=jax.ShapeDtypeStruct((n, dout), jnp.float32),
        grid=(n // r,),
        in_specs=[
            dtop,
            dbot,
            stop,
            sbot,
            pl.BlockSpec((r, din), lambda i: (i, 0)),
            pl.BlockSpec((din, dout), lambda i: (0, 0)),
            pl.BlockSpec((1, dout), lambda i: (0, 0)),
        ],
        out_specs=pl.BlockSpec((r, dout), lambda i: (i, 0)),
    )(deg2, deg2, s2, s2, y, w, b)


def kernel(x, edge_index, W1, b1, W_mu, b_mu, W_ls, b_ls):
    n, din = x.shape
    e = edge_index.shape[1]
    dout = W_mu.shape[1]
    n_pad = ((n + 127) // 128) * 128
    r = 1000 if n % 1000 == 0 else 500

    src = edge_index[0]
    dst = edge_index[1]
    row_zeros = jnp.zeros((n_pad, din), jnp.float32)
    k = _chunk_size(e // NW)
    ones_pat = jnp.zeros((k, din), jnp.float32).at[:, 0].set(1.0)

    deg2 = _deg_pass(n_pad, e, din)(dst, ones_pat, row_zeros)

    y1 = _scale_call(n, din, r, deg2, x)
    s1 = _edge_pass(n_pad, e, din)(y1, src, dst, row_zeros)
    y2 = _fused_call(
        _fused1_body, n, din, din, r, deg2, s1, y1, W1, b1.reshape(1, -1)
    )
    s2 = _edge_pass(n_pad, e, din)(y2, src, dst, row_zeros)

    wc = jnp.concatenate([W_mu, W_ls], axis=1)
    bc = jnp.concatenate([b_mu, b_ls]).reshape(1, -1)
    out = _fused_call(_fused2_body, n, din, 2 * dout, r, deg2, s2, y2, wc, bc)
    return out[:, :dout], out[:, dout:]


# speedup vs baseline: 30.7605x; 2.1316x over previous
import functools

import jax
import jax.numpy as jnp
from jax import lax
from jax.experimental import pallas as pl
from jax.experimental.pallas import tpu as pltpu
from jax.experimental.pallas import tpu_sc as plsc

NC = 2
NS = 16
NW = NC * NS


def _chunk_size(epw):
    for k in (128, 80, 64, 40, 16, 8):
        if epw % k == 0:
            return k
    raise ValueError(f"edges-per-worker {epw} not divisible by a valid chunk")


@functools.lru_cache(maxsize=None)
def _deg_pass(n_pad, e, d):
    epw = e // NW
    k = _chunk_size(epw)
    nchunk = epw // k
    rpt = n_pad // NS

    mesh = plsc.VectorSubcoreMesh(core_axis_name="c", subcore_axis_name="s")

    @functools.partial(
        pl.kernel,
        out_type=jax.ShapeDtypeStruct((NC, n_pad, d), jnp.float32),
        mesh=mesh,
        scratch_types=[
            pltpu.VMEM((nchunk, k), jnp.int32),
            pltpu.VMEM((k, d), jnp.float32),
            pltpu.VMEM_SHARED((n_pad, d), jnp.float32),
            pltpu.SemaphoreType.DMA,
        ],
    )
    def deg_kernel(
        dst_hbm, ones_hbm, zeros_hbm, out_hbm, dst_v, ones_v, acc_sh, ssem
    ):
        cid = lax.axis_index("c")
        sid = lax.axis_index("s")
        wid = cid * NS + sid

        pltpu.sync_copy(ones_hbm, ones_v)

        r0 = sid * rpt
        pltpu.sync_copy(dst_hbm.at[wid], dst_v)
        pltpu.sync_copy(zeros_hbm.at[pl.ds(r0, rpt)], acc_sh.at[pl.ds(r0, rpt)])
        plsc.subcore_barrier()

        grp = 8

        @pl.loop(0, nchunk // grp)
        def _body(j):
            for t in range(grp):
                pltpu.async_copy(
                    ones_v, acc_sh.at[dst_v.at[j * grp + t]], ssem, add=True
                )
            for t in range(grp):
                pltpu.make_async_copy(
                    ones_v, acc_sh.at[dst_v.at[j * grp + t]], ssem
                ).wait()

        for ci in range(nchunk - nchunk % grp, nchunk):
            pltpu.sync_copy(ones_v, acc_sh.at[dst_v.at[ci]], add=True)

        plsc.subcore_barrier()
        pltpu.sync_copy(
            acc_sh.at[pl.ds(r0, rpt)], out_hbm.at[cid, pl.ds(r0, rpt)]
        )

    return deg_kernel


@functools.lru_cache(maxsize=None)
def _edge_pass(n_pad, e, d):
    epw = e // NW
    k = _chunk_size(epw)
    nchunk = epw // k
    rpt = n_pad // NS

    mesh = plsc.VectorSubcoreMesh(core_axis_name="c", subcore_axis_name="s")

    @functools.partial(
        pl.kernel,
        out_type=jax.ShapeDtypeStruct((NC, n_pad, d), jnp.float32),
        mesh=mesh,
        scratch_types=[
            pltpu.VMEM((epw,), jnp.int32),
            pltpu.VMEM((nchunk, k), jnp.int32),
            pltpu.VMEM((k, d), jnp.float32),
            pltpu.VMEM((k, d), jnp.float32),
            pltpu.VMEM_SHARED((n_pad, d), jnp.float32),
            pltpu.SemaphoreType.DMA,
            pltpu.SemaphoreType.DMA,
        ],
    )
    def pass_kernel(
        y_hbm, src_hbm, dst_hbm, zeros_hbm, out_hbm,
        src_v, dst_v, rows0, rows1, acc_sh, sem0, sem1,
    ):
        cid = lax.axis_index("c")
        sid = lax.axis_index("s")
        wid = cid * NS + sid

        r0 = sid * rpt
        base = pl.multiple_of(wid * epw, 8)
        pltpu.sync_copy(src_hbm.at[pl.ds(base, epw)], src_v)
        pltpu.sync_copy(dst_hbm.at[wid], dst_v)
        pltpu.sync_copy(zeros_hbm.at[pl.ds(r0, rpt)], acc_sh.at[pl.ds(r0, rpt)])
        plsc.subcore_barrier()

        def start_gather(ci, buf, sem):
            pltpu.async_copy(y_hbm.at[src_v.at[pl.ds(ci * k, k)]], buf, sem)

        def wait_gather(ci, buf, sem):
            pltpu.make_async_copy(
                y_hbm.at[src_v.at[pl.ds(ci * k, k)]], buf, sem
            ).wait()

        def finish(ci, buf, sem):
            wait_gather(ci, buf, sem)
            pltpu.sync_copy(buf, acc_sh.at[dst_v.at[ci]], add=True)

        start_gather(0, rows0, sem0)

        @pl.loop(0, nchunk // 2)
        def _body(j):
            ci = 2 * j
            start_gather(ci + 1, rows1, sem1)
            finish(ci, rows0, sem0)

            @pl.when(ci + 2 < nchunk)
            def _():
                start_gather(ci + 2, rows0, sem0)

            finish(ci + 1, rows1, sem1)

        if nchunk % 2 == 1:
            finish(nchunk - 1, rows0, sem0)

        plsc.subcore_barrier()
        pltpu.sync_copy(
            acc_sh.at[pl.ds(r0, rpt)], out_hbm.at[cid, pl.ds(r0, rpt)]
        )

    return pass_kernel


def _dinv_block(deg_a, deg_b):
    return lax.rsqrt(deg_a[0, :, 0:1] + deg_b[0, :, 0:1] + 1.0)


def _scale_body(deg_a, deg_b, x_ref, o_ref):
    o_ref[...] = _dinv_block(deg_a[...], deg_b[...]) * x_ref[...]


def _fused1_body(deg_a, deg_b, s_a, s_b, y_ref, w_ref, b_ref, o_ref):
    dinv = _dinv_block(deg_a[...], deg_b[...])
    agg = dinv * (s_a[0] + s_b[0] + y_ref[...])
    h = (
        jnp.dot(
            agg,
            w_ref[...],
            precision=lax.Precision.HIGHEST,
            preferred_element_type=jnp.float32,
        )
        + b_ref[...]
    )
    o_ref[...] = dinv * jnp.maximum(h, 0.0)


def _fused2_body(deg_a, deg_b, s_a, s_b, y_ref, w_ref, b_ref, o1_ref, o2_ref):
    dinv = _dinv_block(deg_a[...], deg_b[...])
    agg = dinv * (s_a[0] + s_b[0] + y_ref[...])
    out = (
        jnp.dot(
            agg,
            w_ref[...],
            precision=lax.Precision.HIGHEST,
            preferred_element_type=jnp.float32,
        )
        + b_ref[...]
    )
    half = o1_ref.shape[1]
    o1_ref[...] = out[:, :half]
    o2_ref[...] = out[:, half:]


def _core_blocks(r, width):
    top = pl.BlockSpec((1, r, width), lambda i: (0, i, 0))
    bot = pl.BlockSpec((1, r, width), lambda i: (1, i, 0))
    return top, bot


def _scale_call(n, din, r, deg2, x):
    dtop, dbot = _core_blocks(r, din)
    return pl.pallas_call(
        _scale_body,
        out_shape=jax.ShapeDtypeStruct((n, din), jnp.float32),
        grid=(n // r,),
        in_specs=[
            dtop,
            dbot,
            pl.BlockSpec((r, din), lambda i: (i, 0)),
        ],
        out_specs=pl.BlockSpec((r, din), lambda i: (i, 0)),
    )(deg2, deg2, x)


def _fused_call(body, n, din, dout, r, deg2, s2, y, w, b, n_out=1):
    dtop, dbot = _core_blocks(r, din)
    stop, sbot = _core_blocks(r, din)
    if n_out == 1:
        out_shape = jax.ShapeDtypeStruct((n, dout), jnp.float32)
        out_specs = pl.BlockSpec((r, dout), lambda i: (i, 0))
    else:
        half = dout // 2
        out_shape = [
            jax.ShapeDtypeStruct((n, half), jnp.float32),
            jax.ShapeDtypeStruct((n, half), jnp.float32),
        ]
        out_specs = [
            pl.BlockSpec((r, half), lambda i: (i, 0)),
            pl.BlockSpec((r, half), lambda i: (i, 0)),
        ]
    return pl.pallas_call(
        body,
        out_shape=out_shape,
        grid=(n // r,),
        in_specs=[
            dtop,
            dbot,
            stop,
            sbot,
            pl.BlockSpec((r, din), lambda i: (i, 0)),
            pl.BlockSpec((din, dout), lambda i: (0, 0)),
            pl.BlockSpec((1, dout), lambda i: (0, 0)),
        ],
        out_specs=out_specs,
    )(deg2, deg2, s2, s2, y, w, b)


def kernel(x, edge_index, W1, b1, W_mu, b_mu, W_ls, b_ls):
    n, din = x.shape
    e = edge_index.shape[1]
    dout = W_mu.shape[1]
    n_pad = ((n + 127) // 128) * 128
    r = 2000 if n % 2000 == 0 else (1000 if n % 1000 == 0 else 500)

    epw = e // NW
    k3 = _chunk_size(epw)
    src = edge_index[0]
    dst = edge_index[1].reshape(NW, epw // k3, k3)
    row_zeros = jnp.zeros((n_pad, din), jnp.float32)
    ones_pat = jnp.zeros((k3, din), jnp.float32).at[:, 0].set(1.0)

    deg2 = _deg_pass(n_pad, e, din)(dst, ones_pat, row_zeros)

    y1 = _scale_call(n, din, r, deg2, x)
    s1 = _edge_pass(n_pad, e, din)(y1, src, dst, row_zeros)
    y2 = _fused_call(
        _fused1_body, n, din, din, r, deg2, s1, y1, W1, b1.reshape(1, -1)
    )
    s2 = _edge_pass(n_pad, e, din)(y2, src, dst, row_zeros)

    wc = jnp.concatenate([W_mu, W_ls], axis=1)
    bc = jnp.concatenate([b_mu, b_ls]).reshape(1, -1)
    mu, ls = _fused_call(
        _fused2_body, n, din, 2 * dout, r, deg2, s2, y2, wc, bc, n_out=2
    )
    return mu, ls


# speedup vs baseline: 30.7883x; 1.0009x over previous
import functools

import jax
import jax.numpy as jnp
from jax import lax
from jax.experimental import pallas as pl
from jax.experimental.pallas import tpu as pltpu
from jax.experimental.pallas import tpu_sc as plsc

NC = 2
NS = 16
NW = NC * NS


def _chunk_size(epw):
    for k in (128, 80, 64, 40, 16, 8):
        if epw % k == 0:
            return k
    raise ValueError(f"edges-per-worker {epw} not divisible by a valid chunk")


@functools.lru_cache(maxsize=None)
def _deg_pass(n_pad, e, d):
    epw = e // NW
    k = _chunk_size(epw)
    nchunk = epw // k
    rpt = n_pad // NS

    mesh = plsc.VectorSubcoreMesh(core_axis_name="c", subcore_axis_name="s")

    @functools.partial(
        pl.kernel,
        out_type=jax.ShapeDtypeStruct((NC, n_pad, d), jnp.float32),
        mesh=mesh,
        scratch_types=[
            pltpu.VMEM((nchunk, k), jnp.int32),
            pltpu.VMEM((k, d), jnp.float32),
            pltpu.VMEM_SHARED((n_pad, d), jnp.float32),
            pltpu.SemaphoreType.DMA,
        ],
    )
    def deg_kernel(
        dst_hbm, ones_hbm, zeros_hbm, out_hbm, dst_v, ones_v, acc_sh, ssem
    ):
        cid = lax.axis_index("c")
        sid = lax.axis_index("s")
        wid = cid * NS + sid

        pltpu.sync_copy(ones_hbm, ones_v)

        r0 = sid * rpt
        pltpu.sync_copy(dst_hbm.at[wid], dst_v)
        pltpu.sync_copy(zeros_hbm.at[pl.ds(r0, rpt)], acc_sh.at[pl.ds(r0, rpt)])
        plsc.subcore_barrier()

        grp = 16

        @pl.loop(0, nchunk // grp)
        def _body(j):
            for t in range(grp):
                pltpu.async_copy(
                    ones_v, acc_sh.at[dst_v.at[j * grp + t]], ssem, add=True
                )
            for t in range(grp):
                pltpu.make_async_copy(
                    ones_v, acc_sh.at[dst_v.at[j * grp + t]], ssem
                ).wait()

        for ci in range(nchunk - nchunk % grp, nchunk):
            pltpu.sync_copy(ones_v, acc_sh.at[dst_v.at[ci]], add=True)

        plsc.subcore_barrier()
        pltpu.sync_copy(
            acc_sh.at[pl.ds(r0, rpt)], out_hbm.at[cid, pl.ds(r0, rpt)]
        )

    return deg_kernel


@functools.lru_cache(maxsize=None)
def _edge_pass(n_pad, e, d):
    epw = e // NW
    k = _chunk_size(epw)
    nchunk = epw // k
    rpt = n_pad // NS

    mesh = plsc.VectorSubcoreMesh(core_axis_name="c", subcore_axis_name="s")

    @functools.partial(
        pl.kernel,
        out_type=jax.ShapeDtypeStruct((NC, n_pad, d), jnp.float32),
        mesh=mesh,
        scratch_types=[
            pltpu.VMEM((epw,), jnp.int32),
            pltpu.VMEM((nchunk, k), jnp.int32),
            pltpu.VMEM((k, d), jnp.float32),
            pltpu.VMEM((k, d), jnp.float32),
            pltpu.VMEM_SHARED((n_pad, d), jnp.float32),
            pltpu.SemaphoreType.DMA,
            pltpu.SemaphoreType.DMA,
        ],
    )
    def pass_kernel(
        y_hbm, src_hbm, dst_hbm, zeros_hbm, out_hbm,
        src_v, dst_v, rows0, rows1, acc_sh, sem0, sem1,
    ):
        cid = lax.axis_index("c")
        sid = lax.axis_index("s")
        wid = cid * NS + sid

        r0 = sid * rpt
        base = pl.multiple_of(wid * epw, 8)
        pltpu.sync_copy(src_hbm.at[pl.ds(base, epw)], src_v)
        pltpu.sync_copy(dst_hbm.at[wid], dst_v)
        pltpu.sync_copy(zeros_hbm.at[pl.ds(r0, rpt)], acc_sh.at[pl.ds(r0, rpt)])
        plsc.subcore_barrier()

        def start_gather(ci, buf, sem):
            pltpu.async_copy(y_hbm.at[src_v.at[pl.ds(ci * k, k)]], buf, sem)

        def wait_gather(ci, buf, sem):
            pltpu.make_async_copy(
                y_hbm.at[src_v.at[pl.ds(ci * k, k)]], buf, sem
            ).wait()

        def finish(ci, buf, sem):
            wait_gather(ci, buf, sem)
            pltpu.sync_copy(buf, acc_sh.at[dst_v.at[ci]], add=True)

        start_gather(0, rows0, sem0)

        @pl.loop(0, nchunk // 2)
        def _body(j):
            ci = 2 * j
            start_gather(ci + 1, rows1, sem1)
            finish(ci, rows0, sem0)

            @pl.when(ci + 2 < nchunk)
            def _():
                start_gather(ci + 2, rows0, sem0)

            finish(ci + 1, rows1, sem1)

        if nchunk % 2 == 1:
            finish(nchunk - 1, rows0, sem0)

        plsc.subcore_barrier()
        pltpu.sync_copy(
            acc_sh.at[pl.ds(r0, rpt)], out_hbm.at[cid, pl.ds(r0, rpt)]
        )

    return pass_kernel


def _dinv_block(deg_a, deg_b):
    return lax.rsqrt(deg_a[0, :, 0:1] + deg_b[0, :, 0:1] + 1.0)


def _scale_body(deg_a, deg_b, x_ref, o_ref):
    o_ref[...] = _dinv_block(deg_a[...], deg_b[...]) * x_ref[...]


def _fused1_body(deg_a, deg_b, s_a, s_b, y_ref, w_ref, b_ref, o_ref):
    dinv = _dinv_block(deg_a[...], deg_b[...])
    agg = dinv * (s_a[0] + s_b[0] + y_ref[...])
    h = (
        jnp.dot(
            agg,
            w_ref[...],
            precision=lax.Precision.HIGHEST,
            preferred_element_type=jnp.float32,
        )
        + b_ref[...]
    )
    o_ref[...] = dinv * jnp.maximum(h, 0.0)


def _fused2_body(deg_a, deg_b, s_a, s_b, y_ref, w_ref, b_ref, o1_ref, o2_ref):
    dinv = _dinv_block(deg_a[...], deg_b[...])
    agg = dinv * (s_a[0] + s_b[0] + y_ref[...])
    out = (
        jnp.dot(
            agg,
            w_ref[...],
            precision=lax.Precision.HIGHEST,
            preferred_element_type=jnp.float32,
        )
        + b_ref[...]
    )
    half = o1_ref.shape[1]
    o1_ref[...] = out[:, :half]
    o2_ref[...] = out[:, half:]


def _core_blocks(r, width):
    top = pl.BlockSpec((1, r, width), lambda i: (0, i, 0))
    bot = pl.BlockSpec((1, r, width), lambda i: (1, i, 0))
    return top, bot


def _scale_call(n, din, r, deg2, x):
    dtop, dbot = _core_blocks(r, din)
    return pl.pallas_call(
        _scale_body,
        out_shape=jax.ShapeDtypeStruct((n, din), jnp.float32),
        grid=(n // r,),
        in_specs=[
            dtop,
            dbot,
            pl.BlockSpec((r, din), lambda i: (i, 0)),
        ],
        out_specs=pl.BlockSpec((r, din), lambda i: (i, 0)),
    )(deg2, deg2, x)


def _fused_call(body, n, din, dout, r, deg2, s2, y, w, b, n_out=1):
    dtop, dbot = _core_blocks(r, din)
    stop, sbot = _core_blocks(r, din)
    if n_out == 1:
        out_shape = jax.ShapeDtypeStruct((n, dout), jnp.float32)
        out_specs = pl.BlockSpec((r, dout), lambda i: (i, 0))
    else:
        half = dout // 2
        out_shape = [
            jax.ShapeDtypeStruct((n, half), jnp.float32),
            jax.ShapeDtypeStruct((n, half), jnp.float32),
        ]
        out_specs = [
            pl.BlockSpec((r, half), lambda i: (i, 0)),
            pl.BlockSpec((r, half), lambda i: (i, 0)),
        ]
    return pl.pallas_call(
        body,
        out_shape=out_shape,
        grid=(n // r,),
        in_specs=[
            dtop,
            dbot,
            stop,
            sbot,
            pl.BlockSpec((r, din), lambda i: (i, 0)),
            pl.BlockSpec((din, dout), lambda i: (0, 0)),
            pl.BlockSpec((1, dout), lambda i: (0, 0)),
        ],
        out_specs=out_specs,
    )(deg2, deg2, s2, s2, y, w, b)


def kernel(x, edge_index, W1, b1, W_mu, b_mu, W_ls, b_ls):
    n, din = x.shape
    e = edge_index.shape[1]
    dout = W_mu.shape[1]
    n_pad = ((n + 127) // 128) * 128
    r = 2000 if n % 2000 == 0 else (1000 if n % 1000 == 0 else 500)

    epw = e // NW
    k3 = _chunk_size(epw)
    src = edge_index[0]
    dst = edge_index[1].reshape(NW, epw // k3, k3)
    row_zeros = jnp.zeros((n_pad, din), jnp.float32)
    ones_pat = jnp.zeros((k3, din), jnp.float32).at[:, 0].set(1.0)

    deg2 = _deg_pass(n_pad, e, din)(dst, ones_pat, row_zeros)

    y1 = _scale_call(n, din, r, deg2, x)
    s1 = _edge_pass(n_pad, e, din)(y1, src, dst, row_zeros)
    y2 = _fused_call(
        _fused1_body, n, din, din, r, deg2, s1, y1, W1, b1.reshape(1, -1)
    )
    s2 = _edge_pass(n_pad, e, din)(y2, src, dst, row_zeros)

    wc = jnp.concatenate([W_mu, W_ls], axis=1)
    bc = jnp.concatenate([b_mu, b_ls]).reshape(1, -1)
    mu, ls = _fused_call(
        _fused2_body, n, din, 2 * dout, r, deg2, s2, y2, wc, bc, n_out=2
    )
    return mu, ls
